# Initial kernel scaffold; baseline (speedup 1.0000x reference)
import jax
import jax.numpy as jnp
from jax.experimental import pallas as pl


def kernel(hidden_states, weight):
    raise NotImplementedError("write your pallas kernel here")


# speedup vs baseline: 1.0624x; 1.0624x over previous
import jax
import jax.numpy as jnp
from jax.experimental import pallas as pl
from jax.experimental.pallas import tpu as pltpu

N_EXP = 64
K = 8
ROW_BLK = 512


def _gate_kernel(x_ref, w_ref, idx_ref, wgt_ref):
    x = x_ref[...]
    w = w_ref[...]
    logits = jax.lax.dot_general(
        x, w, (((1,), (1,)), ((), ())),
        preferred_element_type=jnp.float32)

    m = jnp.max(logits, axis=1, keepdims=True)
    e = jnp.exp(logits - m)
    p = e / jnp.sum(e, axis=1, keepdims=True)

    iota = jax.lax.broadcasted_iota(jnp.int32, p.shape, 1)
    vals, idxs = [], []
    s = p
    for _ in range(K):
        mv = jnp.max(s, axis=1, keepdims=True)
        mi = jnp.min(jnp.where(s == mv, iota, N_EXP), axis=1, keepdims=True)
        vals.append(mv)
        idxs.append(mi)
        s = jnp.where(iota == mi, -1.0, s)

    tw = jnp.concatenate(vals, axis=1)
    ti = jnp.concatenate(idxs, axis=1)
    tw = tw / (jnp.sum(tw, axis=1, keepdims=True) + 1e-20)
    idx_ref[...] = ti
    wgt_ref[...] = tw


def kernel(hidden_states, weight):
    bsz, seq, h = hidden_states.shape
    n = bsz * seq
    hs = hidden_states.reshape(n, h)
    grid = (n // ROW_BLK,)
    ti, tw = pl.pallas_call(
        _gate_kernel,
        grid=grid,
        in_specs=[
            pl.BlockSpec((ROW_BLK, h), lambda i: (i, 0)),
            pl.BlockSpec((N_EXP, h), lambda i: (0, 0)),
        ],
        out_specs=[
            pl.BlockSpec((ROW_BLK, K), lambda i: (i, 0)),
            pl.BlockSpec((ROW_BLK, K), lambda i: (i, 0)),
        ],
        out_shape=[
            jax.ShapeDtypeStruct((n, K), jnp.int32),
            jax.ShapeDtypeStruct((n, K), jnp.float32),
        ],
        compiler_params=pltpu.CompilerParams(
            dimension_semantics=("arbitrary",),
        ),
    )(hs, weight)
    return ti, tw


# speedup vs baseline: 2.0747x; 1.9528x over previous
import jax
import jax.numpy as jnp
from jax.experimental import pallas as pl
from jax.experimental.pallas import tpu as pltpu

N_EXP = 64
K = 8
ROW_BLK = 512


def _gate_kernel(x_ref, w_ref, idx_ref, wgt_ref):
    x = x_ref[...]
    w = w_ref[...]
    lt = jax.lax.dot_general(
        w, x, (((1,), (1,)), ((), ())),
        preferred_element_type=jnp.float32)

    m = jnp.max(lt, axis=0, keepdims=True)
    e = jnp.exp(lt - m)
    p = e / jnp.sum(e, axis=0, keepdims=True)

    iota = jax.lax.broadcasted_iota(jnp.int32, p.shape, 0).astype(jnp.float32)
    vals, idxs = [], []
    s = p
    for _ in range(K):
        mv = jnp.max(s, axis=0, keepdims=True)
        mi = jnp.min(jnp.where(s == mv, iota, float(N_EXP)),
                     axis=0, keepdims=True)
        vals.append(mv)
        idxs.append(mi)
        s = jnp.where(iota == mi, -1.0, s)

    tw = jnp.concatenate(vals, axis=0)
    ti = jnp.concatenate(idxs, axis=0).astype(jnp.int32)
    tw = tw / (jnp.sum(tw, axis=0, keepdims=True) + 1e-20)
    idx_ref[...] = ti
    wgt_ref[...] = tw


def kernel(hidden_states, weight):
    bsz, seq, h = hidden_states.shape
    n = bsz * seq
    hs = hidden_states.reshape(n, h)
    grid = (n // ROW_BLK,)
    ti, tw = pl.pallas_call(
        _gate_kernel,
        grid=grid,
        in_specs=[
            pl.BlockSpec((ROW_BLK, h), lambda i: (i, 0)),
            pl.BlockSpec((N_EXP, h), lambda i: (0, 0)),
        ],
        out_specs=[
            pl.BlockSpec((K, ROW_BLK), lambda i: (0, i)),
            pl.BlockSpec((K, ROW_BLK), lambda i: (0, i)),
        ],
        out_shape=[
            jax.ShapeDtypeStruct((K, n), jnp.int32),
            jax.ShapeDtypeStruct((K, n), jnp.float32),
        ],
        compiler_params=pltpu.CompilerParams(
            dimension_semantics=("arbitrary",),
        ),
    )(hs, weight)
    return ti.T, tw.T


# speedup vs baseline: 2.5133x; 1.2114x over previous
import jax
import jax.numpy as jnp
from jax.experimental import pallas as pl
from jax.experimental.pallas import tpu as pltpu

N_EXP = 64
K = 8
ROW_BLK = 1024


def _gate_kernel(x_ref, w_ref, idx_ref, wgt_ref):
    x = x_ref[...]
    w = w_ref[...]
    lt = jax.lax.dot_general(
        w, x, (((1,), (1,)), ((), ())),
        preferred_element_type=jnp.float32)

    m = jnp.max(lt, axis=0, keepdims=True)
    e = jnp.exp(lt - m)
    p = e / jnp.sum(e, axis=0, keepdims=True)

    iota = jax.lax.broadcasted_iota(jnp.int32, p.shape, 0).astype(jnp.float32)
    vals, idxs = [], []
    s = p
    for _ in range(K):
        mv = jnp.max(s, axis=0, keepdims=True)
        mi = jnp.min(jnp.where(s == mv, iota, float(N_EXP)),
                     axis=0, keepdims=True)
        vals.append(mv)
        idxs.append(mi)
        s = jnp.where(iota == mi, -1.0, s)

    tw = jnp.concatenate(vals, axis=0)
    ti = jnp.concatenate(idxs, axis=0).astype(jnp.int32)
    tw = tw / (jnp.sum(tw, axis=0, keepdims=True) + 1e-20)
    idx_ref[...] = ti
    wgt_ref[...] = tw


def kernel(hidden_states, weight):
    bsz, seq, h = hidden_states.shape
    n = bsz * seq
    hs = hidden_states.reshape(n, h)
    grid = (n // ROW_BLK,)
    ti, tw = pl.pallas_call(
        _gate_kernel,
        grid=grid,
        in_specs=[
            pl.BlockSpec((ROW_BLK, h), lambda i: (i, 0)),
            pl.BlockSpec((N_EXP, h), lambda i: (0, 0)),
        ],
        out_specs=[
            pl.BlockSpec((K, ROW_BLK), lambda i: (0, i)),
            pl.BlockSpec((K, ROW_BLK), lambda i: (0, i)),
        ],
        out_shape=[
            jax.ShapeDtypeStruct((K, n), jnp.int32),
            jax.ShapeDtypeStruct((K, n), jnp.float32),
        ],
        compiler_params=pltpu.CompilerParams(
            dimension_semantics=("arbitrary",),
        ),
    )(hs, weight)
    return ti.T, tw.T


# speedup vs baseline: 2.6599x; 1.0583x over previous
import jax
import jax.numpy as jnp
from jax.experimental import pallas as pl
from jax.experimental.pallas import tpu as pltpu

N_EXP = 64
K = 8
ROW_BLK = 2048


def _gate_kernel(x_ref, w_ref, idx_ref, wgt_ref):
    x = x_ref[...]
    w = w_ref[...]
    lt = jax.lax.dot_general(
        w, x, (((1,), (1,)), ((), ())),
        preferred_element_type=jnp.float32)

    m = jnp.max(lt, axis=0, keepdims=True)
    e = jnp.exp(lt - m)
    p = e / jnp.sum(e, axis=0, keepdims=True)

    iota = jax.lax.broadcasted_iota(jnp.int32, p.shape, 0).astype(jnp.float32)
    vals, idxs = [], []
    s = p
    for _ in range(K):
        mv = jnp.max(s, axis=0, keepdims=True)
        mi = jnp.min(jnp.where(s == mv, iota, float(N_EXP)),
                     axis=0, keepdims=True)
        vals.append(mv)
        idxs.append(mi)
        s = jnp.where(iota == mi, -1.0, s)

    tw = jnp.concatenate(vals, axis=0)
    ti = jnp.concatenate(idxs, axis=0).astype(jnp.int32)
    tw = tw / (jnp.sum(tw, axis=0, keepdims=True) + 1e-20)
    idx_ref[...] = ti
    wgt_ref[...] = tw


def kernel(hidden_states, weight):
    bsz, seq, h = hidden_states.shape
    n = bsz * seq
    hs = hidden_states.reshape(n, h)
    grid = (n // ROW_BLK,)
    ti, tw = pl.pallas_call(
        _gate_kernel,
        grid=grid,
        in_specs=[
            pl.BlockSpec((ROW_BLK, h), lambda i: (i, 0)),
            pl.BlockSpec((N_EXP, h), lambda i: (0, 0)),
        ],
        out_specs=[
            pl.BlockSpec((K, ROW_BLK), lambda i: (0, i)),
            pl.BlockSpec((K, ROW_BLK), lambda i: (0, i)),
        ],
        out_shape=[
            jax.ShapeDtypeStruct((K, n), jnp.int32),
            jax.ShapeDtypeStruct((K, n), jnp.float32),
        ],
        compiler_params=pltpu.CompilerParams(
            dimension_semantics=("arbitrary",),
        ),
    )(hs, weight)
    return ti.T, tw.T
